# baseline (speedup 1.0000x reference)
import jax
import jax.numpy as jnp
from jax.experimental import pallas as pl
from jax.experimental.pallas import tpu as pltpu

D_MODEL = 1024
NUM_FREQ = 3
ROWS_PER_BLOCK = 1024


def _body(idx_ref, x_ref, emb_ref, o_ref):
    idx = idx_ref[0]
    rows = emb_ref[...]
    sel = jax.lax.broadcasted_iota(jnp.int32, (NUM_FREQ, 1), 0) == idx
    row = jnp.sum(jnp.where(sel, rows, 0.0), axis=0, keepdims=True)
    o_ref[...] = x_ref[...] + row


def kernel(x, freq_idx, freq_embeddings):
    orig_shape = x.shape
    d = orig_shape[-1]
    x2 = x.reshape(-1, d)
    n_rows = x2.shape[0]
    rpb = ROWS_PER_BLOCK
    grid = (n_rows // rpb,)
    idx_arr = jnp.asarray(freq_idx, jnp.int32).reshape(1)

    grid_spec = pltpu.PrefetchScalarGridSpec(
        num_scalar_prefetch=1,
        grid=grid,
        in_specs=[
            pl.BlockSpec((rpb, d), lambda i, s: (i, 0)),
            pl.BlockSpec((NUM_FREQ, d), lambda i, s: (0, 0)),
        ],
        out_specs=pl.BlockSpec((rpb, d), lambda i, s: (i, 0)),
    )
    out = pl.pallas_call(
        _body,
        grid_spec=grid_spec,
        out_shape=jax.ShapeDtypeStruct((n_rows, d), x.dtype),
        compiler_params=pltpu.CompilerParams(
            dimension_semantics=("arbitrary",),
        ),
    )(idx_arr, x2, freq_embeddings)
    return out.reshape(orig_shape)


# speedup vs baseline: 1.0114x; 1.0114x over previous
import jax
import jax.numpy as jnp
from jax.experimental import pallas as pl
from jax.experimental.pallas import tpu as pltpu

D_MODEL = 1024
NUM_FREQ = 3
ROWS_PER_BLOCK = 2048


def _body(idx_ref, x_ref, emb_ref, o_ref):
    idx = idx_ref[0]
    rows = emb_ref[...]
    sel = jax.lax.broadcasted_iota(jnp.int32, (NUM_FREQ, 1), 0) == idx
    row = jnp.sum(jnp.where(sel, rows, 0.0), axis=0, keepdims=True)
    o_ref[...] = x_ref[...] + row


def kernel(x, freq_idx, freq_embeddings):
    orig_shape = x.shape
    d = orig_shape[-1]
    x2 = x.reshape(-1, d)
    n_rows = x2.shape[0]
    rpb = ROWS_PER_BLOCK
    grid = (n_rows // rpb,)
    idx_arr = jnp.asarray(freq_idx, jnp.int32).reshape(1)

    grid_spec = pltpu.PrefetchScalarGridSpec(
        num_scalar_prefetch=1,
        grid=grid,
        in_specs=[
            pl.BlockSpec((rpb, d), lambda i, s: (i, 0)),
            pl.BlockSpec((NUM_FREQ, d), lambda i, s: (0, 0)),
        ],
        out_specs=pl.BlockSpec((rpb, d), lambda i, s: (i, 0)),
    )
    out = pl.pallas_call(
        _body,
        grid_spec=grid_spec,
        out_shape=jax.ShapeDtypeStruct((n_rows, d), x.dtype),
        compiler_params=pltpu.CompilerParams(
            dimension_semantics=("arbitrary",),
        ),
    )(idx_arr, x2, freq_embeddings)
    return out.reshape(orig_shape)
